# baseline (speedup 1.0000x reference)
import functools
import math

import jax
import jax.numpy as jnp
from jax import lax
from jax.experimental import pallas as pl
from jax.experimental.pallas import tpu as pltpu
from jax.experimental.pallas import tpu_sc as plsc

NC = 2
NS = 16
NW = NC * NS

CH = 128
NBUF = 12


def _sc_gather_scale_seq(table, tok, scale):
    B, S = tok.shape
    V, D = table.shape
    nsq = B // NW
    nouter = -(-nsq // NBUF)
    nprime = min(NBUF - 1, nsq)
    mesh = plsc.VectorSubcoreMesh(core_axis_name="c", subcore_axis_name="s")

    @functools.partial(
        pl.kernel,
        out_type=jax.ShapeDtypeStruct((B, S, D), jnp.float32),
        mesh=mesh,
        scratch_types=[
            pltpu.VMEM((nsq, S), jnp.int32),
            pltpu.VMEM((NBUF, S, D), jnp.float32),
            pltpu.SemaphoreType.DMA((NBUF,)),
            pltpu.SemaphoreType.DMA((NBUF,)),
        ],
    )
    def run(table_hbm, tok_hbm, out_hbm, idx_v, rows_v, gsems, wsems):
        wid = lax.axis_index("s") * NC + lax.axis_index("c")
        sbase = wid * nsq
        pltpu.sync_copy(tok_hbm.at[pl.ds(sbase, nsq)], idx_v)
        for b in range(nprime):
            pltpu.async_copy(
                table_hbm.at[idx_v.at[b]], rows_v.at[b], gsems.at[b]
            )

        def outer(g, carry):
            for b in range(NBUF):
                q = g * NBUF + b

                @pl.when(q < nsq)
                def _():
                    pltpu.make_async_copy(
                        table_hbm.at[idx_v.at[0]], rows_v.at[b], gsems.at[b]
                    ).wait()

                    def mrow(r, c_):
                        for c in range(D // 16):
                            rows_v[b, r, pl.ds(c * 16, 16)] = (
                                rows_v[b, r, pl.ds(c * 16, 16)] * scale
                            )
                        return c_

                    lax.fori_loop(0, S, mrow, 0)

                    pltpu.async_copy(
                        rows_v.at[b], out_hbm.at[sbase + q], wsems.at[b]
                    )

                    qn = q + NBUF - 1
                    bp = (b + NBUF - 1) % NBUF

                    @pl.when(qn < nsq)
                    def _():
                        @pl.when(q >= 1)
                        def _():
                            pltpu.make_async_copy(
                                rows_v.at[bp], out_hbm.at[sbase], wsems.at[bp]
                            ).wait()

                        pltpu.async_copy(
                            table_hbm.at[idx_v.at[qn]], rows_v.at[bp],
                            gsems.at[bp],
                        )
            return carry

        lax.fori_loop(0, nouter, outer, 0)

        for b in range(min(NBUF, nsq)):
            pltpu.make_async_copy(
                rows_v.at[b], out_hbm.at[sbase], wsems.at[b]
            ).wait()

    return run(table, tok)


def _sc_gather_scale_flat(table, idx, scale):
    NP, = idx.shape
    V, D = table.shape
    npw = NP // NW
    nchunk = npw // CH
    nouter = -(-nchunk // NBUF)
    nbuf = min(NBUF, 7)
    nprime = min(nbuf - 1, nchunk)
    mesh = plsc.VectorSubcoreMesh(core_axis_name="c", subcore_axis_name="s")

    @functools.partial(
        pl.kernel,
        out_type=jax.ShapeDtypeStruct((NP, D), jnp.float32),
        mesh=mesh,
        scratch_types=[
            pltpu.VMEM((npw,), jnp.int32),
            pltpu.VMEM((nbuf, CH, D), jnp.float32),
            pltpu.SemaphoreType.DMA((nbuf,)),
            pltpu.SemaphoreType.DMA((nbuf,)),
        ],
    )
    def run(table_hbm, idx_hbm, out_hbm, idx_v, rows_v, gsems, wsems):
        wid = lax.axis_index("s") * NC + lax.axis_index("c")
        rbase = wid * npw
        pltpu.sync_copy(idx_hbm.at[pl.ds(rbase, npw)], idx_v)
        for b in range(nprime):
            pltpu.async_copy(
                table_hbm.at[idx_v.at[pl.ds(b * CH, CH)]], rows_v.at[b], gsems.at[b]
            )

        def outer(g, carry):
            for b in range(nbuf):
                j = g * nbuf + b

                @pl.when(j < nchunk)
                def _():
                    pltpu.make_async_copy(
                        table_hbm.at[idx_v.at[pl.ds(0, CH)]], rows_v.at[b], gsems.at[b]
                    ).wait()

                    def mrow(r, c_):
                        for c in range(D // 16):
                            rows_v[b, r, pl.ds(c * 16, 16)] = (
                                rows_v[b, r, pl.ds(c * 16, 16)] * scale
                            )
                        return c_

                    lax.fori_loop(0, CH, mrow, 0)

                    pltpu.async_copy(
                        rows_v.at[b], out_hbm.at[pl.ds(rbase + j * CH, CH)],
                        wsems.at[b],
                    )
                    jn = j + nbuf - 1
                    bp = (b + nbuf - 1) % nbuf

                    @pl.when(jn < nchunk)
                    def _():
                        @pl.when(j >= 1)
                        def _():
                            pltpu.make_async_copy(
                                rows_v.at[bp], out_hbm.at[pl.ds(rbase, CH)],
                                wsems.at[bp],
                            ).wait()

                        pltpu.async_copy(
                            table_hbm.at[idx_v.at[pl.ds(jn * CH, CH)]],
                            rows_v.at[bp], gsems.at[bp],
                        )
            return carry

        lax.fori_loop(0, nouter, outer, 0)

        for b in range(min(nbuf, nchunk)):
            pltpu.make_async_copy(
                rows_v.at[b], out_hbm.at[pl.ds(rbase, CH)], wsems.at[b]
            ).wait()

    return run(table, idx)


def kernel(tokens, embedding):
    B, S = tokens.shape
    V, D = embedding.shape
    N = B * S
    scale = math.sqrt(D)
    if B % (2 * NW) == 0 and S <= 128 and D % 16 == 0:
        tok = tokens.astype(jnp.int32)
        bh = B // 2
        h1 = _sc_gather_scale_seq(embedding, lax.slice(tok, (0, 0), (bh, S)), scale)
        h2 = _sc_gather_scale_seq(embedding, lax.slice(tok, (bh, 0), (B, S)), scale)
        return jnp.concatenate([h1, h2], axis=0)
    if B % NW == 0 and S <= 128 and D % 16 == 0:
        return _sc_gather_scale_seq(embedding, tokens.astype(jnp.int32), scale)
    idx = tokens.reshape(N).astype(jnp.int32)
    span = NW * CH
    NP = -(-N // span) * span
    if NP != N:
        idx = jnp.concatenate([idx, jnp.zeros((NP - N,), jnp.int32)])
    out = _sc_gather_scale_flat(embedding, idx, scale)
    if NP != N:
        out = out[:N]
    return out.reshape(B, S, D)


# speedup vs baseline: 1.6059x; 1.6059x over previous
import functools
import math

import jax
import jax.numpy as jnp
from jax import lax
from jax.experimental import pallas as pl
from jax.experimental.pallas import tpu as pltpu
from jax.experimental.pallas import tpu_sc as plsc

NC = 2
NS = 16
NW = NC * NS

CH = 128
NBUF = 12


def _sc_gather_scale_seq(table, tok, scale):
    B, S = tok.shape
    V, D = table.shape
    nsq = B // NW
    nouter = -(-nsq // NBUF)
    nprime = min(NBUF - 1, nsq)
    mesh = plsc.VectorSubcoreMesh(core_axis_name="c", subcore_axis_name="s")

    @functools.partial(
        pl.kernel,
        out_type=jax.ShapeDtypeStruct((B, S, D), jnp.float32),
        mesh=mesh,
        scratch_types=[
            pltpu.VMEM((nsq, S), jnp.int32),
            pltpu.VMEM((NBUF, S, D), jnp.float32),
            pltpu.SemaphoreType.DMA((NBUF,)),
            pltpu.SemaphoreType.DMA((NBUF,)),
        ],
    )
    def run(table_hbm, tok_hbm, out_hbm, idx_v, rows_v, gsems, wsems):
        wid = lax.axis_index("s") * NC + lax.axis_index("c")
        sbase = wid * nsq
        pltpu.sync_copy(tok_hbm.at[pl.ds(sbase, nsq)], idx_v)
        for b in range(nprime):
            pltpu.async_copy(
                table_hbm.at[idx_v.at[b]], rows_v.at[b], gsems.at[b]
            )

        def outer(g, carry):
            for b in range(NBUF):
                q = g * NBUF + b

                @pl.when(q < nsq)
                def _():
                    pltpu.make_async_copy(
                        table_hbm.at[idx_v.at[0]], rows_v.at[b], gsems.at[b]
                    ).wait()

                    def mrow(r, c_):
                        for c in range(D // 16):
                            rows_v[b, r, pl.ds(c * 16, 16)] = (
                                rows_v[b, r, pl.ds(c * 16, 16)] * scale
                            )
                        return c_

                    lax.fori_loop(0, S, mrow, 0)

                    pltpu.async_copy(
                        rows_v.at[b], out_hbm.at[sbase + q], wsems.at[b]
                    )

                    qn = q + NBUF - 1
                    bp = (b + NBUF - 1) % NBUF

                    @pl.when(qn < nsq)
                    def _():
                        @pl.when(q >= 1)
                        def _():
                            pltpu.make_async_copy(
                                rows_v.at[bp], out_hbm.at[sbase], wsems.at[bp]
                            ).wait()

                        pltpu.async_copy(
                            table_hbm.at[idx_v.at[qn]], rows_v.at[bp],
                            gsems.at[bp],
                        )
            return carry

        lax.fori_loop(0, nouter, outer, 0)

        for b in range(min(NBUF, nsq)):
            pltpu.make_async_copy(
                rows_v.at[b], out_hbm.at[sbase], wsems.at[b]
            ).wait()

    return run(table, tok)


def _sc_gather_scale_flat(table, idx, scale):
    NP, = idx.shape
    V, D = table.shape
    npw = NP // NW
    nchunk = npw // CH
    nouter = -(-nchunk // NBUF)
    nbuf = min(NBUF, 7)
    nprime = min(nbuf - 1, nchunk)
    mesh = plsc.VectorSubcoreMesh(core_axis_name="c", subcore_axis_name="s")

    @functools.partial(
        pl.kernel,
        out_type=jax.ShapeDtypeStruct((NP, D), jnp.float32),
        mesh=mesh,
        scratch_types=[
            pltpu.VMEM((npw,), jnp.int32),
            pltpu.VMEM((nbuf, CH, D), jnp.float32),
            pltpu.SemaphoreType.DMA((nbuf,)),
            pltpu.SemaphoreType.DMA((nbuf,)),
        ],
    )
    def run(table_hbm, idx_hbm, out_hbm, idx_v, rows_v, gsems, wsems):
        wid = lax.axis_index("s") * NC + lax.axis_index("c")
        rbase = wid * npw
        pltpu.sync_copy(idx_hbm.at[pl.ds(rbase, npw)], idx_v)
        for b in range(nprime):
            pltpu.async_copy(
                table_hbm.at[idx_v.at[pl.ds(b * CH, CH)]], rows_v.at[b], gsems.at[b]
            )

        def outer(g, carry):
            for b in range(nbuf):
                j = g * nbuf + b

                @pl.when(j < nchunk)
                def _():
                    pltpu.make_async_copy(
                        table_hbm.at[idx_v.at[pl.ds(0, CH)]], rows_v.at[b], gsems.at[b]
                    ).wait()

                    def mrow(r, c_):
                        for c in range(D // 16):
                            rows_v[b, r, pl.ds(c * 16, 16)] = (
                                rows_v[b, r, pl.ds(c * 16, 16)] * scale
                            )
                        return c_

                    lax.fori_loop(0, CH, mrow, 0)

                    pltpu.async_copy(
                        rows_v.at[b], out_hbm.at[pl.ds(rbase + j * CH, CH)],
                        wsems.at[b],
                    )
                    jn = j + nbuf - 1
                    bp = (b + nbuf - 1) % nbuf

                    @pl.when(jn < nchunk)
                    def _():
                        @pl.when(j >= 1)
                        def _():
                            pltpu.make_async_copy(
                                rows_v.at[bp], out_hbm.at[pl.ds(rbase, CH)],
                                wsems.at[bp],
                            ).wait()

                        pltpu.async_copy(
                            table_hbm.at[idx_v.at[pl.ds(jn * CH, CH)]],
                            rows_v.at[bp], gsems.at[bp],
                        )
            return carry

        lax.fori_loop(0, nouter, outer, 0)

        for b in range(min(nbuf, nchunk)):
            pltpu.make_async_copy(
                rows_v.at[b], out_hbm.at[pl.ds(rbase, CH)], wsems.at[b]
            ).wait()

    return run(table, idx)


def kernel(tokens, embedding):
    B, S = tokens.shape
    V, D = embedding.shape
    N = B * S
    scale = math.sqrt(D)
    if B % NW == 0 and S <= 128 and D % 16 == 0:
        return _sc_gather_scale_seq(embedding, tokens.astype(jnp.int32), scale)
    idx = tokens.reshape(N).astype(jnp.int32)
    span = NW * CH
    NP = -(-N // span) * span
    if NP != N:
        idx = jnp.concatenate([idx, jnp.zeros((NP - N,), jnp.int32)])
    out = _sc_gather_scale_flat(embedding, idx, scale)
    if NP != N:
        out = out[:N]
    return out.reshape(B, S, D)
